# baseline (speedup 1.0000x reference)
import functools

import jax
import jax.numpy as jnp
from jax.experimental import pallas as pl
from jax.experimental.pallas import tpu as pltpu

_B, _C, _T, _K = 16, 512, 512, 4096
_N = _B * _T
_R = 256
_TB = _T // _R
_STEPS = _N // _R


def _vq_kernel(f_ref, t_ref, cb_ref, loss_ref, acc_ref, y2_ref):
    f = f_ref[0]
    cb = cb_ref[...]

    b = pl.program_id(0)
    tb = pl.program_id(1)

    @pl.when(jnp.logical_and(b == 0, tb == 0))
    def _fill_y2():
        y2_ref[...] = jnp.sum(cb * cb, axis=1, keepdims=True)

    dot = jax.lax.dot_general(
        cb, f, (((1,), (0,)), ((), ())),
        preferred_element_type=jnp.float32)

    y2 = y2_ref[...]
    score = y2 - 2.0 * dot

    t_row = t_ref[0]
    kio = jax.lax.broadcasted_iota(jnp.int32, (_K, _R), 0)
    mask_t = kio == t_row
    score_t = jnp.sum(jnp.where(mask_t, score, 0.0),
                      axis=0, keepdims=True)
    smin = jnp.min(score, axis=0, keepdims=True)

    x2 = jnp.sum(f * f, axis=0, keepdims=True)
    loss_ref[...] = jnp.sum(x2 + score_t).reshape(1, 1, 1)
    acc_ref[...] = jnp.sum(
        (score_t <= smin).astype(jnp.float32)).reshape(1, 1, 1)


@functools.partial(jax.jit, static_argnames=())
def kernel(student_features, teacher_codes, codebook, distance_matrix):
    del distance_matrix
    teacher = teacher_codes.reshape(_B, 1, _T).astype(jnp.int32)

    loss_p, acc_p = pl.pallas_call(
        _vq_kernel,
        grid=(_B, _TB),
        in_specs=[
            pl.BlockSpec((1, _C, _R), lambda b, tb: (b, 0, tb)),
            pl.BlockSpec((1, 1, _R), lambda b, tb: (b, 0, tb)),
            pl.BlockSpec((_K, _C), lambda b, tb: (0, 0)),
        ],
        out_specs=[
            pl.BlockSpec((1, 1, 1), lambda b, tb: (b * _TB + tb, 0, 0)),
            pl.BlockSpec((1, 1, 1), lambda b, tb: (b * _TB + tb, 0, 0)),
        ],
        out_shape=[
            jax.ShapeDtypeStruct((_STEPS, 1, 1), jnp.float32),
            jax.ShapeDtypeStruct((_STEPS, 1, 1), jnp.float32),
        ],
        scratch_shapes=[pltpu.VMEM((_K, 1), jnp.float32)],
    )(student_features, teacher, codebook)

    loss = jnp.sum(loss_p) / float(_N * _C)
    accuracy = jnp.sum(acc_p) / float(_N)
    return (loss, accuracy)


# speedup vs baseline: 1.1068x; 1.1068x over previous
import functools

import jax
import jax.numpy as jnp
from jax.experimental import pallas as pl

_B, _C, _T, _K = 16, 512, 512, 4096
_N = _B * _T
_R = 512
_TB = _T // _R
_STEPS = _N // _R


def _vq_kernel(f_ref, t_ref, cb_ref, loss_ref, acc_ref):
    f = f_ref[0]
    cb = cb_ref[...]

    dot = jax.lax.dot_general(
        cb, f, (((1,), (0,)), ((), ())),
        preferred_element_type=jnp.float32)

    y2 = jnp.sum(cb * cb, axis=1, keepdims=True)
    score = y2 - 2.0 * dot

    t_row = t_ref[0]
    kio = jax.lax.broadcasted_iota(jnp.int32, (_K, _R), 0)
    mask_t = kio == t_row
    score_t = jnp.sum(jnp.where(mask_t, score, 0.0),
                      axis=0, keepdims=True)
    smin = jnp.min(score, axis=0, keepdims=True)

    x2 = jnp.sum(f * f, axis=0, keepdims=True)
    loss_ref[...] = jnp.sum(x2 + score_t).reshape(1, 1, 1)
    acc_ref[...] = jnp.sum(
        (score_t <= smin).astype(jnp.float32)).reshape(1, 1, 1)


@functools.partial(jax.jit, static_argnames=())
def kernel(student_features, teacher_codes, codebook, distance_matrix):
    del distance_matrix
    teacher = teacher_codes.reshape(_B, 1, _T).astype(jnp.int32)

    loss_p, acc_p = pl.pallas_call(
        _vq_kernel,
        grid=(_B, _TB),
        in_specs=[
            pl.BlockSpec((1, _C, _R), lambda b, tb: (b, 0, tb)),
            pl.BlockSpec((1, 1, _R), lambda b, tb: (b, 0, tb)),
            pl.BlockSpec((_K, _C), lambda b, tb: (0, 0)),
        ],
        out_specs=[
            pl.BlockSpec((1, 1, 1), lambda b, tb: (b * _TB + tb, 0, 0)),
            pl.BlockSpec((1, 1, 1), lambda b, tb: (b * _TB + tb, 0, 0)),
        ],
        out_shape=[
            jax.ShapeDtypeStruct((_STEPS, 1, 1), jnp.float32),
            jax.ShapeDtypeStruct((_STEPS, 1, 1), jnp.float32),
        ],
    )(student_features, teacher, codebook)

    loss = jnp.sum(loss_p) / float(_N * _C)
    accuracy = jnp.sum(acc_p) / float(_N)
    return (loss, accuracy)


# speedup vs baseline: 1.1094x; 1.0024x over previous
import functools

import jax
import jax.numpy as jnp
from jax.experimental import pallas as pl
from jax.experimental.pallas import tpu as pltpu

_B, _C, _T, _K = 16, 512, 512, 4096
_N = _B * _T
_R = 512
_TB = _T // _R
_STEPS = _N // _R


def _vq_kernel(f_ref, t_ref, cb_ref, loss_ref, acc_ref):
    f = f_ref[0]
    cb = cb_ref[...]

    dot = jax.lax.dot_general(
        cb, f, (((1,), (0,)), ((), ())),
        preferred_element_type=jnp.float32)

    y2 = jnp.sum(cb * cb, axis=1, keepdims=True)
    score = y2 - 2.0 * dot

    t_row = t_ref[0]
    kio = jax.lax.broadcasted_iota(jnp.int32, (_K, _R), 0)
    mask_t = kio == t_row
    score_t = jnp.sum(jnp.where(mask_t, score, 0.0),
                      axis=0, keepdims=True)
    smin = jnp.min(score, axis=0, keepdims=True)

    x2 = jnp.sum(f * f, axis=0, keepdims=True)
    loss_ref[...] = jnp.sum(x2 + score_t).reshape(1, 1, 1)
    acc_ref[...] = jnp.sum(
        (score_t <= smin).astype(jnp.float32)).reshape(1, 1, 1)


@functools.partial(jax.jit, static_argnames=())
def kernel(student_features, teacher_codes, codebook, distance_matrix):
    del distance_matrix
    teacher = teacher_codes.reshape(_B, 1, _T).astype(jnp.int32)

    loss_p, acc_p = pl.pallas_call(
        _vq_kernel,
        grid=(_B, _TB),
        in_specs=[
            pl.BlockSpec((1, _C, _R), lambda b, tb: (b, 0, tb)),
            pl.BlockSpec((1, 1, _R), lambda b, tb: (b, 0, tb)),
            pl.BlockSpec((_K, _C), lambda b, tb: (0, 0)),
        ],
        out_specs=[
            pl.BlockSpec((1, 1, 1), lambda b, tb: (b * _TB + tb, 0, 0)),
            pl.BlockSpec((1, 1, 1), lambda b, tb: (b * _TB + tb, 0, 0)),
        ],
        out_shape=[
            jax.ShapeDtypeStruct((_STEPS, 1, 1), jnp.float32),
            jax.ShapeDtypeStruct((_STEPS, 1, 1), jnp.float32),
        ],
        compiler_params=pltpu.CompilerParams(
            dimension_semantics=("parallel", "arbitrary")),
    )(student_features, teacher, codebook)

    loss = jnp.sum(loss_p) / float(_N * _C)
    accuracy = jnp.sum(acc_p) / float(_N)
    return (loss, accuracy)


# speedup vs baseline: 1.1950x; 1.0771x over previous
import functools

import jax
import jax.numpy as jnp
from jax.experimental import pallas as pl
from jax.experimental.pallas import tpu as pltpu

_B, _C, _T, _K = 16, 512, 512, 4096
_N = _B * _T
_R = 512
_TB = _T // _R
_STEPS = _N // _R


def _y2_kernel(cb_ref, y2_ref):
    cb = cb_ref[...]
    y2_ref[...] = jnp.sum(cb * cb, axis=1, keepdims=True)


def _vq_kernel(f_ref, t_ref, y2_ref, cb_ref, loss_ref, acc_ref):
    f = f_ref[0]
    cb = cb_ref[...]

    x2 = jnp.sum(f * f, axis=0, keepdims=True)
    fm2 = -2.0 * f
    dot = jax.lax.dot_general(
        cb, fm2, (((1,), (0,)), ((), ())),
        preferred_element_type=jnp.float32)

    score = y2_ref[...] + dot

    t_row = t_ref[0]
    kio = jax.lax.broadcasted_iota(jnp.int32, (_K, _R), 0)
    mask_t = kio == t_row
    score_t = jnp.sum(jnp.where(mask_t, score, 0.0),
                      axis=0, keepdims=True)
    smin = jnp.min(score, axis=0, keepdims=True)

    loss_ref[...] = jnp.sum(x2 + score_t).reshape(1, 1, 1)
    acc_ref[...] = jnp.sum(
        (score_t <= smin).astype(jnp.float32)).reshape(1, 1, 1)


@functools.partial(jax.jit, static_argnames=())
def kernel(student_features, teacher_codes, codebook, distance_matrix):
    del distance_matrix
    teacher = teacher_codes.reshape(_B, 1, _T).astype(jnp.int32)

    y2 = pl.pallas_call(
        _y2_kernel,
        out_shape=jax.ShapeDtypeStruct((_K, 1), jnp.float32),
    )(codebook)

    loss_p, acc_p = pl.pallas_call(
        _vq_kernel,
        grid=(_B, _TB),
        in_specs=[
            pl.BlockSpec((1, _C, _R), lambda b, tb: (b, 0, tb)),
            pl.BlockSpec((1, 1, _R), lambda b, tb: (b, 0, tb)),
            pl.BlockSpec((_K, 1), lambda b, tb: (0, 0)),
            pl.BlockSpec((_K, _C), lambda b, tb: (0, 0)),
        ],
        out_specs=[
            pl.BlockSpec((1, 1, 1), lambda b, tb: (b * _TB + tb, 0, 0)),
            pl.BlockSpec((1, 1, 1), lambda b, tb: (b * _TB + tb, 0, 0)),
        ],
        out_shape=[
            jax.ShapeDtypeStruct((_STEPS, 1, 1), jnp.float32),
            jax.ShapeDtypeStruct((_STEPS, 1, 1), jnp.float32),
        ],
        compiler_params=pltpu.CompilerParams(
            dimension_semantics=("arbitrary", "arbitrary")),
    )(student_features, teacher, y2, codebook)

    loss = jnp.sum(loss_p) / float(_N * _C)
    accuracy = jnp.sum(acc_p) / float(_N)
    return (loss, accuracy)


# speedup vs baseline: 1.2591x; 1.0537x over previous
import functools

import jax
import jax.numpy as jnp
from jax.experimental import pallas as pl
from jax.experimental.pallas import tpu as pltpu

_B, _C, _T, _K = 16, 512, 512, 4096
_N = _B * _T
_R = 512
_STEPS = _N // _R
_KB = 8


def _y2_kernel(cb_ref, y2_ref):
    cb = cb_ref[...]
    y2_ref[...] = jnp.sum(cb * cb, axis=1, keepdims=True)


def _vq_kernel(f_ref, t_ref, y2_ref, cb_ref, loss_ref, acc_ref):
    i = pl.program_id(0)
    f = f_ref[0]
    cb = cb_ref[...]

    x2 = jnp.sum(f * f, axis=0, keepdims=True)
    fm2 = -2.0 * f
    dot = jax.lax.dot_general(
        cb, fm2, (((1,), (0,)), ((), ())),
        preferred_element_type=jnp.float32)

    score = y2_ref[...] + dot

    t_row = t_ref[0]
    kio = jax.lax.broadcasted_iota(jnp.int32, (_K, _R), 0)
    mask_t = kio == t_row
    score_t = jnp.sum(jnp.where(mask_t, score, 0.0),
                      axis=0, keepdims=True)
    smin = jnp.min(score, axis=0, keepdims=True)

    block_loss = jnp.sum(x2 + score_t).reshape(1, 1)
    block_match = jnp.sum(
        (score_t <= smin).astype(jnp.float32)).reshape(1, 1)

    @pl.when(i == 0)
    def _init():
        loss_ref[...] = jnp.zeros((1, 1), jnp.float32)
        acc_ref[...] = jnp.zeros((1, 1), jnp.float32)

    loss_ref[...] += block_loss
    acc_ref[...] += block_match

    @pl.when(i == _STEPS - 1)
    def _norm():
        loss_ref[...] = loss_ref[...] * (1.0 / float(_N * _C))
        acc_ref[...] = acc_ref[...] * (1.0 / float(_N))


@functools.partial(jax.jit, static_argnames=())
def kernel(student_features, teacher_codes, codebook, distance_matrix):
    del distance_matrix
    teacher = teacher_codes.reshape(_B, 1, _T).astype(jnp.int32)

    y2 = pl.pallas_call(
        _y2_kernel,
        grid=(_KB,),
        in_specs=[pl.BlockSpec((_K // _KB, _C), lambda i: (i, 0))],
        out_specs=pl.BlockSpec((_K // _KB, 1), lambda i: (i, 0)),
        out_shape=jax.ShapeDtypeStruct((_K, 1), jnp.float32),
    )(codebook)

    loss, acc = pl.pallas_call(
        _vq_kernel,
        grid=(_STEPS,),
        in_specs=[
            pl.BlockSpec((1, _C, _R), lambda i: (i, 0, 0)),
            pl.BlockSpec((1, 1, _R), lambda i: (i, 0, 0)),
            pl.BlockSpec((_K, 1), lambda i: (0, 0)),
            pl.BlockSpec((_K, _C), lambda i: (0, 0)),
        ],
        out_specs=[
            pl.BlockSpec((1, 1), lambda i: (0, 0)),
            pl.BlockSpec((1, 1), lambda i: (0, 0)),
        ],
        out_shape=[
            jax.ShapeDtypeStruct((1, 1), jnp.float32),
            jax.ShapeDtypeStruct((1, 1), jnp.float32),
        ],
    )(student_features, teacher, y2, codebook)

    return (loss[0, 0], acc[0, 0])


# speedup vs baseline: 1.4001x; 1.1120x over previous
import functools

import jax
import jax.numpy as jnp
from jax.experimental import pallas as pl
from jax.experimental.pallas import tpu as pltpu

_B, _C, _T, _K = 16, 512, 512, 4096
_N = _B * _T
_R = 512
_STEPS = _N // _R
_KB = 8
_KC = 4


def _y2_kernel(cb_ref, y2_ref):
    cb = cb_ref[...]
    y2_ref[...] = jnp.sum(cb * cb, axis=1, keepdims=True)


def _vq_kernel(f_ref, t_ref, y2_ref, cb_ref, loss_ref, acc_ref):
    i = pl.program_id(0)
    f = f_ref[0]
    cb = cb_ref[...]

    x2 = jnp.sum(f * f, axis=0, keepdims=True)
    fm2 = -2.0 * f

    t_row = t_ref[0]
    kc = _K // _KC
    score_t_parts = []
    smin_parts = []
    for c in range(_KC):
        dot_c = jax.lax.dot_general(
            cb[c * kc:(c + 1) * kc, :], fm2, (((1,), (0,)), ((), ())),
            preferred_element_type=jnp.float32)
        score_c = y2_ref[c * kc:(c + 1) * kc, :] + dot_c
        kio = jax.lax.broadcasted_iota(jnp.int32, (kc, _R), 0) + (c * kc)
        mask_c = kio == t_row
        score_t_parts.append(jnp.sum(jnp.where(mask_c, score_c, 0.0),
                                     axis=0, keepdims=True))
        smin_parts.append(jnp.min(score_c, axis=0, keepdims=True))

    score_t = sum(score_t_parts)
    smin = jnp.minimum(jnp.minimum(smin_parts[0], smin_parts[1]),
                       jnp.minimum(smin_parts[2], smin_parts[3]))

    block_loss = jnp.sum(x2 + score_t).reshape(1, 1)
    block_match = jnp.sum(
        (score_t <= smin).astype(jnp.float32)).reshape(1, 1)

    @pl.when(i == 0)
    def _init():
        loss_ref[...] = jnp.zeros((1, 1), jnp.float32)
        acc_ref[...] = jnp.zeros((1, 1), jnp.float32)

    loss_ref[...] += block_loss
    acc_ref[...] += block_match

    @pl.when(i == _STEPS - 1)
    def _norm():
        loss_ref[...] = loss_ref[...] * (1.0 / float(_N * _C))
        acc_ref[...] = acc_ref[...] * (1.0 / float(_N))


@functools.partial(jax.jit, static_argnames=())
def kernel(student_features, teacher_codes, codebook, distance_matrix):
    del distance_matrix
    teacher = teacher_codes.reshape(_B, 1, _T).astype(jnp.int32)

    y2 = pl.pallas_call(
        _y2_kernel,
        grid=(_KB,),
        in_specs=[pl.BlockSpec((_K // _KB, _C), lambda i: (i, 0))],
        out_specs=pl.BlockSpec((_K // _KB, 1), lambda i: (i, 0)),
        out_shape=jax.ShapeDtypeStruct((_K, 1), jnp.float32),
    )(codebook)

    loss, acc = pl.pallas_call(
        _vq_kernel,
        grid=(_STEPS,),
        in_specs=[
            pl.BlockSpec((1, _C, _R), lambda i: (i, 0, 0)),
            pl.BlockSpec((1, 1, _R), lambda i: (i, 0, 0)),
            pl.BlockSpec((_K, 1), lambda i: (0, 0)),
            pl.BlockSpec((_K, _C), lambda i: (0, 0)),
        ],
        out_specs=[
            pl.BlockSpec((1, 1), lambda i: (0, 0)),
            pl.BlockSpec((1, 1), lambda i: (0, 0)),
        ],
        out_shape=[
            jax.ShapeDtypeStruct((1, 1), jnp.float32),
            jax.ShapeDtypeStruct((1, 1), jnp.float32),
        ],
    )(student_features, teacher, y2, codebook)

    return (loss[0, 0], acc[0, 0])


# speedup vs baseline: 1.4118x; 1.0083x over previous
import functools

import jax
import jax.numpy as jnp
from jax.experimental import pallas as pl
from jax.experimental.pallas import tpu as pltpu

_B, _C, _T, _K = 16, 512, 512, 4096
_N = _B * _T
_R = 512
_STEPS = _N // _R
_KB = 8
_KSPLITS = (1024, 1024, 1024, 768, 256)


def _y2_kernel(cb_ref, y2_ref):
    cb = cb_ref[...]
    y2_ref[...] = jnp.sum(cb * cb, axis=1, keepdims=True)


def _vq_kernel(f_ref, t_ref, y2_ref, cb_ref, loss_ref, acc_ref):
    i = pl.program_id(0)
    f = f_ref[0]
    cb = cb_ref[...]

    x2 = jnp.sum(f * f, axis=0, keepdims=True)
    fm2 = -2.0 * f

    t_row = t_ref[0]
    score_t_parts = []
    smin_parts = []
    base = 0
    for kc in _KSPLITS:
        dot_c = jax.lax.dot_general(
            cb[base:base + kc, :], fm2, (((1,), (0,)), ((), ())),
            preferred_element_type=jnp.float32)
        score_c = y2_ref[base:base + kc, :] + dot_c
        kio = jax.lax.broadcasted_iota(jnp.int32, (kc, _R), 0) + base
        mask_c = kio == t_row
        score_t_parts.append(jnp.sum(jnp.where(mask_c, score_c, 0.0),
                                     axis=0, keepdims=True))
        smin_parts.append(jnp.min(score_c, axis=0, keepdims=True))
        base += kc

    score_t = sum(score_t_parts)
    while len(smin_parts) > 1:
        paired = [jnp.minimum(a, b)
                  for a, b in zip(smin_parts[::2], smin_parts[1::2])]
        if len(smin_parts) % 2:
            paired.append(smin_parts[-1])
        smin_parts = paired
    smin = smin_parts[0]

    block_loss = jnp.sum(x2 + score_t).reshape(1, 1)
    block_match = jnp.sum(
        (score_t <= smin).astype(jnp.float32)).reshape(1, 1)

    @pl.when(i == 0)
    def _init():
        loss_ref[...] = jnp.zeros((1, 1), jnp.float32)
        acc_ref[...] = jnp.zeros((1, 1), jnp.float32)

    loss_ref[...] += block_loss
    acc_ref[...] += block_match

    @pl.when(i == _STEPS - 1)
    def _norm():
        loss_ref[...] = loss_ref[...] * (1.0 / float(_N * _C))
        acc_ref[...] = acc_ref[...] * (1.0 / float(_N))


@functools.partial(jax.jit, static_argnames=())
def kernel(student_features, teacher_codes, codebook, distance_matrix):
    del distance_matrix
    teacher = teacher_codes.reshape(_B, 1, _T).astype(jnp.int32)

    y2 = pl.pallas_call(
        _y2_kernel,
        grid=(_KB,),
        in_specs=[pl.BlockSpec((_K // _KB, _C), lambda i: (i, 0))],
        out_specs=pl.BlockSpec((_K // _KB, 1), lambda i: (i, 0)),
        out_shape=jax.ShapeDtypeStruct((_K, 1), jnp.float32),
    )(codebook)

    loss, acc = pl.pallas_call(
        _vq_kernel,
        grid=(_STEPS,),
        in_specs=[
            pl.BlockSpec((1, _C, _R), lambda i: (i, 0, 0)),
            pl.BlockSpec((1, 1, _R), lambda i: (i, 0, 0)),
            pl.BlockSpec((_K, 1), lambda i: (0, 0)),
            pl.BlockSpec((_K, _C), lambda i: (0, 0)),
        ],
        out_specs=[
            pl.BlockSpec((1, 1), lambda i: (0, 0)),
            pl.BlockSpec((1, 1), lambda i: (0, 0)),
        ],
        out_shape=[
            jax.ShapeDtypeStruct((1, 1), jnp.float32),
            jax.ShapeDtypeStruct((1, 1), jnp.float32),
        ],
    )(student_features, teacher, y2, codebook)

    return (loss[0, 0], acc[0, 0])


# speedup vs baseline: 1.4978x; 1.0610x over previous
import functools

import jax
import jax.numpy as jnp
from jax.experimental import pallas as pl
from jax.experimental.pallas import tpu as pltpu

_B, _C, _T, _K = 16, 512, 512, 4096
_N = _B * _T
_KR = 256
_STEPS = _K // _KR
_CS = _C // _STEPS


def _vq_kernel(f_ref, t_ref, cb_ref, loss_ref, acc_ref, st_ref, sm_ref):
    i = pl.program_id(0)
    cbb = cb_ref[...]
    cbm2 = -2.0 * cbb
    y2c = jnp.sum(cbb * cbb, axis=1, keepdims=True)

    @pl.when(i == 0)
    def _init():
        st_ref[...] = jnp.zeros((_B, _T), jnp.float32)
        sm_ref[...] = jnp.full((_B, _T), jnp.inf, jnp.float32)

    kio = jax.lax.broadcasted_iota(jnp.int32, (_KR, _T), 0)
    base = i * _KR

    xs = jnp.zeros((1, 1), jnp.float32)
    for b in range(_B):
        fb = f_ref[b]
        dot_cb = jax.lax.dot_general(
            cbm2, fb, (((1,), (0,)), ((), ())),
            preferred_element_type=jnp.float32)
        score_cb = y2c + dot_cb

        t_b = t_ref[b:b + 1, :] - base
        mask = kio == t_b
        st_b = jnp.sum(jnp.where(mask, score_cb, 0.0),
                       axis=0, keepdims=True)
        sm_b = jnp.min(score_cb, axis=0, keepdims=True)

        st_ref[b:b + 1, :] += st_b
        sm_ref[b:b + 1, :] = jnp.minimum(sm_ref[b:b + 1, :], sm_b)

        fs = f_ref[b, pl.ds(i * _CS, _CS), :]
        xs += jnp.sum(fs * fs).reshape(1, 1)

    @pl.when(i == 0)
    def _zero_out():
        loss_ref[...] = jnp.zeros((1, 1), jnp.float32)

    loss_ref[...] += xs

    @pl.when(i == _STEPS - 1)
    def _final():
        st = st_ref[...]
        sm = sm_ref[...]
        loss_ref[...] = ((loss_ref[...] + jnp.sum(st).reshape(1, 1))
                         * (1.0 / float(_N * _C)))
        acc_ref[...] = (jnp.sum((st <= sm).astype(jnp.float32))
                        .reshape(1, 1) * (1.0 / float(_N)))


@functools.partial(jax.jit, static_argnames=())
def kernel(student_features, teacher_codes, codebook, distance_matrix):
    del distance_matrix
    teacher = teacher_codes.reshape(_B, _T).astype(jnp.int32)

    loss, acc = pl.pallas_call(
        _vq_kernel,
        grid=(_STEPS,),
        in_specs=[
            pl.BlockSpec((_B, _C, _T), lambda i: (0, 0, 0)),
            pl.BlockSpec((_B, _T), lambda i: (0, 0)),
            pl.BlockSpec((_KR, _C), lambda i: (i, 0)),
        ],
        out_specs=[
            pl.BlockSpec((1, 1), lambda i: (0, 0)),
            pl.BlockSpec((1, 1), lambda i: (0, 0)),
        ],
        out_shape=[
            jax.ShapeDtypeStruct((1, 1), jnp.float32),
            jax.ShapeDtypeStruct((1, 1), jnp.float32),
        ],
        scratch_shapes=[
            pltpu.VMEM((_B, _T), jnp.float32),
            pltpu.VMEM((_B, _T), jnp.float32),
        ],
    )(student_features, teacher, codebook)

    return (loss[0, 0], acc[0, 0])
